# baseline (speedup 1.0000x reference)
import jax
import jax.numpy as jnp
from jax.experimental import pallas as pl
from jax.experimental.pallas import tpu as pltpu

_N = 16384
_C = 1000
_BN = 4096
_NB = _N // _BN
_TR = _BN // 128


def _dl_kernel(x_ref, t_ref, out_ref, acc_ref):
    i = pl.program_id(0)
    x = x_ref[...]
    parts = []
    for j in range(_TR):
        xc = x[:, j * 128:(j + 1) * 128]
        row = jax.lax.broadcasted_iota(jnp.int32, xc.shape, 0)
        mx = jnp.max(xc, axis=0, keepdims=True)
        parts.append(jnp.min(jnp.where(xc == mx, row, _C), axis=0, keepdims=True))
    pred = jnp.concatenate(parts, axis=0)
    tgt = t_ref[...]
    m = (pred != tgt).astype(jnp.float32)
    pf = pred.astype(jnp.float32)
    pm = pf * m
    bn = jnp.sum(m)
    bs1 = jnp.sum(pm)
    bs2 = jnp.sum(pf * pm)

    @pl.when(i == 0)
    def _():
        acc_ref[0] = bn
        acc_ref[1] = bs1
        acc_ref[2] = bs2

    @pl.when(i != 0)
    def _():
        acc_ref[0] += bn
        acc_ref[1] += bs1
        acc_ref[2] += bs2

    @pl.when(i == _NB - 1)
    def _():
        n = acc_ref[0]
        s1 = acc_ref[1]
        s2 = acc_ref[2]
        mean = s1 / n
        var = (s2 - s1 * mean) / (n - 1.0)
        out_ref[0, 0] = 1.0 - jnp.sqrt(var)


def kernel(inputs, targets):
    xt = inputs.T
    t128 = targets.reshape(128, 128)
    out = pl.pallas_call(
        _dl_kernel,
        grid=(_NB,),
        in_specs=[
            pl.BlockSpec((_C, _BN), lambda i: (0, i)),
            pl.BlockSpec((_TR, 128), lambda i: (i, 0)),
        ],
        out_specs=pl.BlockSpec(
            (1, 1), lambda i: (0, 0), memory_space=pltpu.SMEM
        ),
        out_shape=jax.ShapeDtypeStruct((1, 1), jnp.float32),
        scratch_shapes=[pltpu.SMEM((3,), jnp.float32)],
        compiler_params=pltpu.CompilerParams(
            dimension_semantics=("arbitrary",),
        ),
    )(xt, t128)
    return out.reshape(())
